# baseline (speedup 1.0000x reference)
import functools

import jax
import jax.numpy as jnp
from jax import lax
from jax.experimental import pallas as pl
from jax.experimental.pallas import tpu as pltpu
from jax.experimental.pallas import tpu_sc as plsc

N, F, V, D = 16384, 26, 1000000, 16
NC, NS = 2, 16
NW = NC * NS
ROWS_PER_W = N // NW
GROUP = 64
NGROUPS = ROWS_PER_W // GROUP
IDX_PER_G = GROUP * F
CHI = 128
NCHUNK = IDX_PER_G // CHI


FP = 32


def _sc_body(fi8_hbm, fi_hbm, fv_hbm, fvp_hbm, emb_hbm, fo_hbm,
             y2_hbm, yf_hbm,
             idx8_v, idx_v, fvv, fvp, rows_v, fo_v, y2_v, yf_v, sem):
    wid = lax.axis_index("s") * NC + lax.axis_index("c")
    row0 = wid * ROWS_PER_W

    def group_body(g, carry):
        base = (row0 + g * GROUP) * F
        pltpu.sync_copy(fi8_hbm.at[pl.ds(base, IDX_PER_G)], idx8_v)
        pltpu.sync_copy(fi_hbm.at[pl.ds(base, IDX_PER_G)], idx_v)
        pltpu.sync_copy(fv_hbm.at[pl.ds(base, IDX_PER_G)], fvv)
        pltpu.sync_copy(
            fvp_hbm.at[pl.ds((row0 + g * GROUP) * FP, GROUP * FP)], fvp)
        handles = []
        for j in range(NCHUNK):
            sl = pl.ds(j * CHI, CHI)
            handles.append(pltpu.async_copy(
                emb_hbm.at[idx8_v.at[sl]], rows_v.at[sl], sem))
            handles.append(pltpu.async_copy(
                fo_hbm.at[idx_v.at[sl]], fo_v.at[sl], sem))
        for h in handles:
            h.wait()

        def yf_body(j, c):
            sl = pl.ds(j * 16, 16)
            yf_v[sl] = fo_v[sl] * fvv[sl]
            return c
        lax.fori_loop(0, IDX_PER_G // 16, yf_body, 0)

        def row_body(r, c):
            rb = r * F
            rp = r * FP
            va = fvp[pl.ds(rp, 16)]
            vb = fvp[pl.ds(rp + 16, 16)]
            s = jnp.zeros((16,), jnp.float32)
            q = jnp.zeros((16,), jnp.float32)
            for f in range(F):
                a = va[f] if f < 16 else vb[f - 16]
                e = rows_v[rb + f, :] * a
                s = s + e
                q = q + e * e
            y2_v[r, :] = 0.5 * (s * s - q)
            return c
        lax.fori_loop(0, GROUP, row_body, 0)

        pltpu.sync_copy(y2_v, y2_hbm.at[pl.ds(row0 + g * GROUP, GROUP)])
        pltpu.sync_copy(yf_v, yf_hbm.at[pl.ds(base, IDX_PER_G)])
        return carry

    lax.fori_loop(0, NGROUPS, group_body, 0)


def _sc_pooling(fi8_flat, fi_flat, fv_flat, fvp_flat, emb_table, fo_flat):
    mesh = plsc.VectorSubcoreMesh(core_axis_name="c", subcore_axis_name="s")
    return pl.kernel(
        _sc_body,
        mesh=mesh,
        compiler_params=pltpu.CompilerParams(use_tc_tiling_on_sc=False),
        out_type=[
            jax.ShapeDtypeStruct((N, D), jnp.float32),
            jax.ShapeDtypeStruct((N * F,), jnp.float32),
        ],
        scratch_types=[
            pltpu.VMEM((IDX_PER_G,), jnp.int32),
            pltpu.VMEM((IDX_PER_G,), jnp.int32),
            pltpu.VMEM((IDX_PER_G,), jnp.float32),
            pltpu.VMEM((GROUP * FP,), jnp.float32),
            pltpu.VMEM((IDX_PER_G, D), jnp.float32),
            pltpu.VMEM((IDX_PER_G,), jnp.float32),
            pltpu.VMEM((GROUP, D), jnp.float32),
            pltpu.VMEM((IDX_PER_G,), jnp.float32),
            pltpu.SemaphoreType.DMA,
        ],
    )(fi8_flat, fi_flat, fv_flat, fvp_flat, emb_table, fo_flat)


REPACK_C = 8192


def _repack_body(a_ref, fo_ref, o_ref, o2_ref, scr):
    scr[...] = jnp.swapaxes(a_ref[...], 0, 1)
    for p in range(2):
        o_ref[:, 64 * p:64 * p + 16] = scr[p::2, :]
    o2_ref[...] = fo_ref[...].reshape(REPACK_C)


def _repack_table(emb_t, fo_t):
    grid = (pl.cdiv(V, REPACK_C),)
    return pl.pallas_call(
        _repack_body,
        grid=grid,
        in_specs=[pl.BlockSpec((D, REPACK_C), lambda i: (0, i)),
                  pl.BlockSpec((1, REPACK_C), lambda i: (0, i))],
        out_specs=[pl.BlockSpec((REPACK_C // 2, 128), lambda i: (i, 0)),
                   pl.BlockSpec((REPACK_C,), lambda i: (i,))],
        out_shape=[jax.ShapeDtypeStruct((V // 2, 128), jnp.float32),
                   jax.ShapeDtypeStruct((V,), jnp.float32)],
        scratch_shapes=[pltpu.VMEM((REPACK_C, D), jnp.float32)],
    )(emb_t, fo_t)


BLK = 2048


def _mlp_body(y2_ref, yf_ref, w0_ref, b0_ref, w1_ref, b1_ref, wout_ref,
              bc_ref, o_ref):
    y2 = y2_ref[...]
    h0 = jnp.maximum(
        jnp.dot(y2, w0_ref[...], preferred_element_type=jnp.float32)
        + b0_ref[...], 0.0)
    h1 = jnp.maximum(
        jnp.dot(h0, w1_ref[...], preferred_element_type=jnp.float32)
        + b1_ref[...], 0.0)
    yd = jnp.dot(h1, wout_ref[...], preferred_element_type=jnp.float32)
    o_ref[...] = yf_ref[...] + (yd + bc_ref[...])


def _mlp_combine(y2, yf2d, W0, b0, W1, b1, Wout, bc):
    grid = (N // BLK,)
    return pl.pallas_call(
        _mlp_body,
        grid=grid,
        in_specs=[
            pl.BlockSpec((BLK, D), lambda i: (i, 0)),
            pl.BlockSpec((BLK, F), lambda i: (i, 0)),
            pl.BlockSpec((D, 64), lambda i: (0, 0)),
            pl.BlockSpec((1, 64), lambda i: (0, 0)),
            pl.BlockSpec((64, 32), lambda i: (0, 0)),
            pl.BlockSpec((1, 32), lambda i: (0, 0)),
            pl.BlockSpec((32, 1), lambda i: (0, 0)),
            pl.BlockSpec((1, 1), lambda i: (0, 0)),
        ],
        out_specs=pl.BlockSpec((BLK, F), lambda i: (i, 0)),
        out_shape=jax.ShapeDtypeStruct((N, F), jnp.float32),
    )(y2, yf2d, W0, b0, W1, b1, Wout, bc)


def kernel(feature_index, feature_value, label, emb_table, fo_table, bias,
           W0, b0, W1, b1, Wout, bout):
    fi_flat = feature_index.reshape(-1).astype(jnp.int32)
    fi8_flat = fi_flat * 4
    fv_flat = feature_value.reshape(-1)
    fvp_flat = jnp.pad(feature_value, ((0, 0), (0, FP - F))).reshape(-1)
    emb_pad, fo_flat = _repack_table(emb_table.T, fo_table.T)
    y2, yf = _sc_pooling(fi8_flat, fi_flat, fv_flat, fvp_flat,
                         emb_pad.reshape(4 * V, D), fo_flat)
    bc = (bias + bout).reshape(1, 1)
    out = _mlp_combine(y2, yf.reshape(N, F), W0, b0.reshape(1, -1),
                       W1, b1.reshape(1, -1), Wout, bc)
    return out


# speedup vs baseline: 1.0895x; 1.0895x over previous
import functools

import jax
import jax.numpy as jnp
from jax import lax
from jax.experimental import pallas as pl
from jax.experimental.pallas import tpu as pltpu
from jax.experimental.pallas import tpu_sc as plsc

N, F, V, D = 16384, 26, 1000000, 16
NC, NS = 2, 16
NW = NC * NS
ROWS_PER_W = N // NW
GROUP = 64
NGROUPS = ROWS_PER_W // GROUP
IDX_PER_G = GROUP * F
CHI = 128
NCHUNK = IDX_PER_G // CHI


FP = 32


def _sc_fo_body(fi_hbm, fv_hbm, fo_hbm, yf_hbm, idx_v, fvv, fo_v, yf_v, sem):
    wid = lax.axis_index("s") * NC + lax.axis_index("c")
    row0 = wid * ROWS_PER_W

    def group_body(g, carry):
        base = (row0 + g * GROUP) * F
        pltpu.sync_copy(fi_hbm.at[pl.ds(base, IDX_PER_G)], idx_v)
        pltpu.sync_copy(fv_hbm.at[pl.ds(base, IDX_PER_G)], fvv)
        handles = [pltpu.async_copy(
            fo_hbm.at[idx_v.at[pl.ds(j * CHI, CHI)]],
            fo_v.at[pl.ds(j * CHI, CHI)], sem) for j in range(NCHUNK)]
        for h in handles:
            h.wait()

        def yf_body(j, c):
            sl = pl.ds(j * 16, 16)
            yf_v[sl] = fo_v[sl] * fvv[sl]
            return c
        lax.fori_loop(0, IDX_PER_G // 16, yf_body, 0)
        pltpu.sync_copy(yf_v, yf_hbm.at[pl.ds(base, IDX_PER_G)])
        return carry

    lax.fori_loop(0, NGROUPS, group_body, 0)


def _sc_first_order(fi_flat, fv_flat, fo_flat):
    mesh = plsc.VectorSubcoreMesh(core_axis_name="c", subcore_axis_name="s")
    return pl.kernel(
        _sc_fo_body,
        mesh=mesh,
        compiler_params=pltpu.CompilerParams(use_tc_tiling_on_sc=False),
        out_type=[jax.ShapeDtypeStruct((N * F,), jnp.float32)],
        scratch_types=[
            pltpu.VMEM((IDX_PER_G,), jnp.int32),
            pltpu.VMEM((IDX_PER_G,), jnp.float32),
            pltpu.VMEM((IDX_PER_G,), jnp.float32),
            pltpu.VMEM((IDX_PER_G,), jnp.float32),
            pltpu.SemaphoreType.DMA,
        ],
    )(fi_flat, fv_flat, fo_flat)[0]


def _sc_body(fi8_hbm, fvp_hbm, emb_hbm, y2_hbm,
             idx8_v, fvp, rows_v, y2_v, sem):
    wid = lax.axis_index("s") * NC + lax.axis_index("c")
    row0 = wid * ROWS_PER_W

    def group_body(g, carry):
        base = (row0 + g * GROUP) * F
        pltpu.sync_copy(fi8_hbm.at[pl.ds(base, IDX_PER_G)], idx8_v)
        pltpu.sync_copy(
            fvp_hbm.at[pl.ds((row0 + g * GROUP) * FP, GROUP * FP)], fvp)
        handles = []
        for j in range(NCHUNK):
            sl = pl.ds(j * CHI, CHI)
            handles.append(pltpu.async_copy(
                emb_hbm.at[idx8_v.at[sl]], rows_v.at[sl], sem))
        for h in handles:
            h.wait()

        def row_body(r, c):
            rb = r * F
            rp = r * FP
            va = fvp[pl.ds(rp, 16)]
            vb = fvp[pl.ds(rp + 16, 16)]
            s = jnp.zeros((16,), jnp.float32)
            q = jnp.zeros((16,), jnp.float32)
            for f in range(F):
                a = va[f] if f < 16 else vb[f - 16]
                e = rows_v[rb + f, :] * a
                s = s + e
                q = q + e * e
            y2_v[r, :] = 0.5 * (s * s - q)
            return c
        lax.fori_loop(0, GROUP, row_body, 0)

        pltpu.sync_copy(y2_v, y2_hbm.at[pl.ds(row0 + g * GROUP, GROUP)])
        return carry

    lax.fori_loop(0, NGROUPS, group_body, 0)


def _sc_pooling(fi8_flat, fvp_flat, emb_table):
    mesh = plsc.VectorSubcoreMesh(core_axis_name="c", subcore_axis_name="s")
    return pl.kernel(
        _sc_body,
        mesh=mesh,
        compiler_params=pltpu.CompilerParams(use_tc_tiling_on_sc=False),
        out_type=[jax.ShapeDtypeStruct((N, D), jnp.float32)],
        scratch_types=[
            pltpu.VMEM((IDX_PER_G,), jnp.int32),
            pltpu.VMEM((GROUP * FP,), jnp.float32),
            pltpu.VMEM((IDX_PER_G, D), jnp.float32),
            pltpu.VMEM((GROUP, D), jnp.float32),
            pltpu.SemaphoreType.DMA,
        ],
    )(fi8_flat, fvp_flat, emb_table)[0]


REPACK_C = 8192


def _repack_body(a_ref, o_ref):
    o_ref[:, 0:16] = jnp.swapaxes(a_ref[...], 0, 1)


def _repack_table(emb_t):
    grid = (pl.cdiv(V, REPACK_C),)
    return pl.pallas_call(
        _repack_body,
        grid=grid,
        in_specs=[pl.BlockSpec((D, REPACK_C), lambda i: (0, i))],
        out_specs=pl.BlockSpec((REPACK_C, 128), lambda i: (i, 0)),
        out_shape=jax.ShapeDtypeStruct((V, 128), jnp.float32),
    )(emb_t)


def _fo_body(a_ref, o_ref):
    o_ref[...] = a_ref[...].reshape(REPACK_C)


def _fo_flatten(fo_t):
    grid = (pl.cdiv(V, REPACK_C),)
    return pl.pallas_call(
        _fo_body,
        grid=grid,
        in_specs=[pl.BlockSpec((1, REPACK_C), lambda i: (0, i))],
        out_specs=pl.BlockSpec((REPACK_C,), lambda i: (i,)),
        out_shape=jax.ShapeDtypeStruct((V,), jnp.float32),
    )(fo_t)


BLK = 2048


def _mlp_body(y2_ref, yf_ref, w0_ref, b0_ref, w1_ref, b1_ref, wout_ref,
              bc_ref, o_ref):
    y2 = y2_ref[...]
    h0 = jnp.maximum(
        jnp.dot(y2, w0_ref[...], preferred_element_type=jnp.float32)
        + b0_ref[...], 0.0)
    h1 = jnp.maximum(
        jnp.dot(h0, w1_ref[...], preferred_element_type=jnp.float32)
        + b1_ref[...], 0.0)
    yd = jnp.dot(h1, wout_ref[...], preferred_element_type=jnp.float32)
    o_ref[...] = yf_ref[...] + (yd + bc_ref[...])


def _mlp_combine(y2, yf2d, W0, b0, W1, b1, Wout, bc):
    grid = (N // BLK,)
    return pl.pallas_call(
        _mlp_body,
        grid=grid,
        in_specs=[
            pl.BlockSpec((BLK, D), lambda i: (i, 0)),
            pl.BlockSpec((BLK, F), lambda i: (i, 0)),
            pl.BlockSpec((D, 64), lambda i: (0, 0)),
            pl.BlockSpec((1, 64), lambda i: (0, 0)),
            pl.BlockSpec((64, 32), lambda i: (0, 0)),
            pl.BlockSpec((1, 32), lambda i: (0, 0)),
            pl.BlockSpec((32, 1), lambda i: (0, 0)),
            pl.BlockSpec((1, 1), lambda i: (0, 0)),
        ],
        out_specs=pl.BlockSpec((BLK, F), lambda i: (i, 0)),
        out_shape=jax.ShapeDtypeStruct((N, F), jnp.float32),
    )(y2, yf2d, W0, b0, W1, b1, Wout, bc)


def kernel(feature_index, feature_value, label, emb_table, fo_table, bias,
           W0, b0, W1, b1, Wout, bout):
    fi_flat = feature_index.reshape(-1).astype(jnp.int32)
    fi8_flat = fi_flat * 8
    fv_flat = feature_value.reshape(-1)
    fvp_flat = jnp.pad(feature_value, ((0, 0), (0, FP - F))).reshape(-1)
    fo_flat = _fo_flatten(fo_table.T)
    yf = _sc_first_order(fi_flat, fv_flat, fo_flat)
    emb_pad = _repack_table(emb_table.T)
    y2 = _sc_pooling(fi8_flat, fvp_flat, emb_pad.reshape(8 * V, D))
    bc = (bias + bout).reshape(1, 1)
    out = _mlp_combine(y2, yf.reshape(N, F), W0, b0.reshape(1, -1),
                       W1, b1.reshape(1, -1), Wout, bc)
    return out


# speedup vs baseline: 1.1893x; 1.0916x over previous
import functools

import jax
import jax.numpy as jnp
from jax import lax
from jax.experimental import pallas as pl
from jax.experimental.pallas import tpu as pltpu
from jax.experimental.pallas import tpu_sc as plsc

N, F, V, D = 16384, 26, 1000000, 16
NC, NS = 2, 16
NW = NC * NS
ROWS_PER_W = N // NW
GROUP = 64
NGROUPS = ROWS_PER_W // GROUP
IDX_PER_G = GROUP * F
CHI = 128
NCHUNK = IDX_PER_G // CHI


FP = 32


def _sc_body(fi8_hbm, fi_hbm, fv_hbm, fvp_hbm, emb_hbm, fo_hbm,
             y2_hbm, yf_hbm,
             idx8_v, idx_v, fvv, fvp, rows_v, fo_v, y2_v, yf_v, sem):
    wid = lax.axis_index("s") * NC + lax.axis_index("c")
    row0 = wid * ROWS_PER_W

    pltpu.sync_copy(fi8_hbm.at[pl.ds(row0 * F, ROWS_PER_W * F)], idx8_v)
    pltpu.sync_copy(fi_hbm.at[pl.ds(row0 * F, ROWS_PER_W * F)], idx_v)
    pltpu.sync_copy(fv_hbm.at[pl.ds(row0 * F, ROWS_PER_W * F)], fvv)
    pltpu.sync_copy(fvp_hbm.at[pl.ds(row0 * FP, ROWS_PER_W * FP)], fvp)

    def group_body(g, carry):
        gb = g * IDX_PER_G
        handles = []
        for j in range(NCHUNK):
            sl = pl.ds(j * CHI, CHI)
            gsl = pl.ds(gb + j * CHI, CHI)
            handles.append(pltpu.async_copy(
                emb_hbm.at[idx8_v.at[gsl]], rows_v.at[sl], sem))
            handles.append(pltpu.async_copy(
                fo_hbm.at[idx_v.at[gsl]], fo_v.at[sl], sem))
        for h in handles:
            h.wait()

        def yf_body(j, c):
            yf_v[pl.ds(gb + j * 16, 16)] = (
                fo_v[pl.ds(j * 16, 16)] * fvv[pl.ds(gb + j * 16, 16)])
            return c
        lax.fori_loop(0, IDX_PER_G // 16, yf_body, 0)

        def row_body(r, c):
            rb = r * F
            rp = g * (GROUP * FP) + r * FP
            va = fvp[pl.ds(rp, 16)]
            vb = fvp[pl.ds(rp + 16, 16)]
            s = jnp.zeros((16,), jnp.float32)
            q = jnp.zeros((16,), jnp.float32)
            for f in range(F):
                a = va[f] if f < 16 else vb[f - 16]
                e = rows_v[rb + f, :] * a
                s = s + e
                q = q + e * e
            y2_v[g * GROUP + r, :] = 0.5 * (s * s - q)
            return c
        lax.fori_loop(0, GROUP, row_body, 0)
        return carry

    lax.fori_loop(0, NGROUPS, group_body, 0)
    pltpu.sync_copy(y2_v, y2_hbm.at[pl.ds(row0, ROWS_PER_W)])
    pltpu.sync_copy(yf_v, yf_hbm.at[pl.ds(row0 * F, ROWS_PER_W * F)])


def _sc_pooling(fi8_flat, fi_flat, fv_flat, fvp_flat, emb_table, fo_flat):
    mesh = plsc.VectorSubcoreMesh(core_axis_name="c", subcore_axis_name="s")
    return pl.kernel(
        _sc_body,
        mesh=mesh,
        compiler_params=pltpu.CompilerParams(use_tc_tiling_on_sc=False),
        out_type=[
            jax.ShapeDtypeStruct((N, D), jnp.float32),
            jax.ShapeDtypeStruct((N * F,), jnp.float32),
        ],
        scratch_types=[
            pltpu.VMEM((ROWS_PER_W * F,), jnp.int32),
            pltpu.VMEM((ROWS_PER_W * F,), jnp.int32),
            pltpu.VMEM((ROWS_PER_W * F,), jnp.float32),
            pltpu.VMEM((ROWS_PER_W * FP,), jnp.float32),
            pltpu.VMEM((IDX_PER_G, D), jnp.float32),
            pltpu.VMEM((IDX_PER_G,), jnp.float32),
            pltpu.VMEM((ROWS_PER_W, D), jnp.float32),
            pltpu.VMEM((ROWS_PER_W * F,), jnp.float32),
            pltpu.SemaphoreType.DMA,
        ],
    )(fi8_flat, fi_flat, fv_flat, fvp_flat, emb_table, fo_flat)


REPACK_C = 8192


def _repack_body(a_ref, fo_ref, o_ref, o2_ref):
    o_ref[:, 0:16] = jnp.swapaxes(a_ref[...], 0, 1)
    o2_ref[...] = fo_ref[...].reshape(REPACK_C)


def _repack_table(emb_t, fo_t):
    grid = (pl.cdiv(V, REPACK_C),)
    return pl.pallas_call(
        _repack_body,
        grid=grid,
        in_specs=[pl.BlockSpec((D, REPACK_C), lambda i: (0, i)),
                  pl.BlockSpec((1, REPACK_C), lambda i: (0, i))],
        out_specs=[pl.BlockSpec((REPACK_C, 128), lambda i: (i, 0)),
                   pl.BlockSpec((REPACK_C,), lambda i: (i,))],
        out_shape=[jax.ShapeDtypeStruct((V, 128), jnp.float32),
                   jax.ShapeDtypeStruct((V,), jnp.float32)],
    )(emb_t, fo_t)


BLK = 2048


def _mlp_body(y2_ref, yf_ref, w0_ref, b0_ref, w1_ref, b1_ref, wout_ref,
              bc_ref, o_ref):
    y2 = y2_ref[...]
    h0 = jnp.maximum(
        jnp.dot(y2, w0_ref[...], preferred_element_type=jnp.float32)
        + b0_ref[...], 0.0)
    h1 = jnp.maximum(
        jnp.dot(h0, w1_ref[...], preferred_element_type=jnp.float32)
        + b1_ref[...], 0.0)
    yd = jnp.dot(h1, wout_ref[...], preferred_element_type=jnp.float32)
    o_ref[...] = yf_ref[...] + (yd + bc_ref[...])


def _mlp_combine(y2, yf2d, W0, b0, W1, b1, Wout, bc):
    grid = (N // BLK,)
    return pl.pallas_call(
        _mlp_body,
        grid=grid,
        in_specs=[
            pl.BlockSpec((BLK, D), lambda i: (i, 0)),
            pl.BlockSpec((BLK, F), lambda i: (i, 0)),
            pl.BlockSpec((D, 64), lambda i: (0, 0)),
            pl.BlockSpec((1, 64), lambda i: (0, 0)),
            pl.BlockSpec((64, 32), lambda i: (0, 0)),
            pl.BlockSpec((1, 32), lambda i: (0, 0)),
            pl.BlockSpec((32, 1), lambda i: (0, 0)),
            pl.BlockSpec((1, 1), lambda i: (0, 0)),
        ],
        out_specs=pl.BlockSpec((BLK, F), lambda i: (i, 0)),
        out_shape=jax.ShapeDtypeStruct((N, F), jnp.float32),
    )(y2, yf2d, W0, b0, W1, b1, Wout, bc)


def kernel(feature_index, feature_value, label, emb_table, fo_table, bias,
           W0, b0, W1, b1, Wout, bout):
    fi_flat = feature_index.reshape(-1).astype(jnp.int32)
    fi8_flat = fi_flat * 8
    fv_flat = feature_value.reshape(-1)
    fvp_flat = jnp.pad(feature_value, ((0, 0), (0, FP - F))).reshape(-1)
    emb_pad, fo_flat = _repack_table(emb_table.T, fo_table.T)
    y2, yf = _sc_pooling(fi8_flat, fi_flat, fv_flat, fvp_flat,
                         emb_pad.reshape(8 * V, D), fo_flat)
    bc = (bias + bout).reshape(1, 1)
    out = _mlp_combine(y2, yf.reshape(N, F), W0, b0.reshape(1, -1),
                       W1, b1.reshape(1, -1), Wout, bc)
    return out
